# baseline (speedup 1.0000x reference)
import jax
import jax.numpy as jnp
from jax.experimental import pallas as pl


def _scale_kernel(x_ref, o_ref):
    ch = jax.lax.broadcasted_iota(jnp.int32, x_ref.shape, 1)
    scale = jnp.where(ch == 0, jnp.float32(1.0), jnp.float32(0.5))
    o_ref[...] = x_ref[...] * scale


def kernel(x):
    B, C, N = x.shape
    return pl.pallas_call(
        _scale_kernel,
        grid=(B,),
        in_specs=[pl.BlockSpec((1, C, N), lambda i: (i, 0, 0))],
        out_specs=pl.BlockSpec((1, C, N), lambda i: (i, 0, 0)),
        out_shape=jax.ShapeDtypeStruct(x.shape, x.dtype),
    )(x)


# speedup vs baseline: 1.0323x; 1.0323x over previous
import jax
import jax.numpy as jnp
from jax.experimental import pallas as pl


def _scale_kernel(x_ref, o_ref):
    ch = jax.lax.broadcasted_iota(jnp.int32, x_ref.shape, 1)
    scale = jnp.where(ch == 0, jnp.float32(1.0), jnp.float32(0.5))
    o_ref[...] = x_ref[...] * scale


def kernel(x):
    B, C, N = x.shape
    BB = 2
    return pl.pallas_call(
        _scale_kernel,
        grid=(B // BB,),
        in_specs=[pl.BlockSpec((BB, C, N), lambda i: (i, 0, 0))],
        out_specs=pl.BlockSpec((BB, C, N), lambda i: (i, 0, 0)),
        out_shape=jax.ShapeDtypeStruct(x.shape, x.dtype),
    )(x)
